# baseline (speedup 1.0000x reference)
import jax
import jax.numpy as jnp
from jax.experimental import pallas as pl
from jax.experimental.pallas import tpu as pltpu


def _nt_dot(x, w):
    return jax.lax.dot_general(
        x.astype(jnp.bfloat16),
        w.astype(jnp.bfloat16),
        (((1,), (1,)), ((), ())),
        preferred_element_type=jnp.float32,
    )


def _mlp_kernel(x_ref, w0_ref, b0_ref, w1_ref, b1_ref, w2_ref, b2_ref, o_ref):
    x = x_ref[0]
    h = jnp.maximum(_nt_dot(x, w0_ref[0]) + b0_ref[0], 0.0)
    h = jnp.maximum(_nt_dot(h, w1_ref[0]) + b1_ref[0], 0.0)
    o_ref[0] = _nt_dot(h, w2_ref[0]) + b2_ref[0]


def kernel(query, W0, b0, W1, b1, W2, b2):
    B, S, D_IN = query.shape
    D_H = W0.shape[1]
    D_OUT = W2.shape[1]
    BS = min(512, S)

    b0r = b0[:, None, :]
    b1r = b1[:, None, :]
    b2r = b2[:, None, :]

    return pl.pallas_call(
        _mlp_kernel,
        out_shape=jax.ShapeDtypeStruct((B, S, D_OUT), query.dtype),
        grid=(B, S // BS),
        in_specs=[
            pl.BlockSpec((1, BS, D_IN), lambda b, s: (b, s, 0)),
            pl.BlockSpec((1, D_H, D_IN), lambda b, s: (b, 0, 0)),
            pl.BlockSpec((1, 1, D_H), lambda b, s: (b, 0, 0)),
            pl.BlockSpec((1, D_H, D_H), lambda b, s: (b, 0, 0)),
            pl.BlockSpec((1, 1, D_H), lambda b, s: (b, 0, 0)),
            pl.BlockSpec((1, D_OUT, D_H), lambda b, s: (b, 0, 0)),
            pl.BlockSpec((1, 1, D_OUT), lambda b, s: (b, 0, 0)),
        ],
        out_specs=pl.BlockSpec((1, BS, D_OUT), lambda b, s: (b, s, 0)),
        compiler_params=pltpu.CompilerParams(
            dimension_semantics=("parallel", "arbitrary"),
            vmem_limit_bytes=56 * 1024 * 1024,
        ),
        name="ltm_mlp",
    )(query, W0, b0r, W1, b1r, W2, b2r)


# speedup vs baseline: 1.1302x; 1.1302x over previous
import jax
import jax.numpy as jnp
from jax.experimental import pallas as pl
from jax.experimental.pallas import tpu as pltpu


def _nt_dot(x, w):
    return jax.lax.dot_general(
        x, w, (((1,), (1,)), ((), ())), preferred_element_type=jnp.float32
    )


def _mlp_kernel(x_ref, w0_ref, b0_ref, w1_ref, b1_ref, w2_ref, b2_ref, o_ref,
                w0b, w1b, w2b):
    @pl.when(pl.program_id(1) == 0)
    def _cast_weights():
        w0b[...] = w0_ref[0].astype(jnp.bfloat16)
        w1b[...] = w1_ref[0].astype(jnp.bfloat16)
        w2b[...] = w2_ref[0].astype(jnp.bfloat16)

    x = x_ref[0].astype(jnp.bfloat16)
    h = jnp.maximum(_nt_dot(x, w0b[...]) + b0_ref[0], 0.0)
    h = jnp.maximum(_nt_dot(h.astype(jnp.bfloat16), w1b[...]) + b1_ref[0], 0.0)
    o_ref[0] = _nt_dot(h.astype(jnp.bfloat16), w2b[...]) + b2_ref[0]


def kernel(query, W0, b0, W1, b1, W2, b2):
    B, S, D_IN = query.shape
    D_H = W0.shape[1]
    D_OUT = W2.shape[1]
    BS = min(1024, S)

    b0r = b0[:, None, :]
    b1r = b1[:, None, :]
    b2r = b2[:, None, :]

    return pl.pallas_call(
        _mlp_kernel,
        out_shape=jax.ShapeDtypeStruct((B, S, D_OUT), query.dtype),
        grid=(B, S // BS),
        in_specs=[
            pl.BlockSpec((1, BS, D_IN), lambda b, s: (b, s, 0)),
            pl.BlockSpec((1, D_H, D_IN), lambda b, s: (b, 0, 0)),
            pl.BlockSpec((1, 1, D_H), lambda b, s: (b, 0, 0)),
            pl.BlockSpec((1, D_H, D_H), lambda b, s: (b, 0, 0)),
            pl.BlockSpec((1, 1, D_H), lambda b, s: (b, 0, 0)),
            pl.BlockSpec((1, D_OUT, D_H), lambda b, s: (b, 0, 0)),
            pl.BlockSpec((1, 1, D_OUT), lambda b, s: (b, 0, 0)),
        ],
        out_specs=pl.BlockSpec((1, BS, D_OUT), lambda b, s: (b, s, 0)),
        scratch_shapes=[
            pltpu.VMEM((D_H, D_IN), jnp.bfloat16),
            pltpu.VMEM((D_H, D_H), jnp.bfloat16),
            pltpu.VMEM((D_OUT, D_H), jnp.bfloat16),
        ],
        compiler_params=pltpu.CompilerParams(
            dimension_semantics=("parallel", "arbitrary"),
            vmem_limit_bytes=56 * 1024 * 1024,
        ),
        name="ltm_mlp",
    )(query, W0, b0r, W1, b1r, W2, b2r)


# speedup vs baseline: 1.1707x; 1.0358x over previous
import jax
import jax.numpy as jnp
from jax.experimental import pallas as pl
from jax.experimental.pallas import tpu as pltpu


def _nt_dot(x, w):
    return jax.lax.dot_general(
        x, w, (((1,), (1,)), ((), ())), preferred_element_type=jnp.float32
    )


def _mlp_kernel(x_ref, w0_ref, b0_ref, w1_ref, b1_ref, w2_ref, b2_ref, o_ref,
                w0b, w1b, w2b):
    @pl.when(pl.program_id(1) == 0)
    def _cast_weights():
        w0b[...] = w0_ref[0].astype(jnp.bfloat16)
        w1b[...] = w1_ref[0].astype(jnp.bfloat16)
        w2b[...] = w2_ref[0].astype(jnp.bfloat16)

    zero = jnp.bfloat16(0)
    x = x_ref[0].astype(jnp.bfloat16)
    h = jnp.maximum((_nt_dot(x, w0b[...]) + b0_ref[0]).astype(jnp.bfloat16), zero)
    h = jnp.maximum((_nt_dot(h, w1b[...]) + b1_ref[0]).astype(jnp.bfloat16), zero)
    o_ref[0] = _nt_dot(h, w2b[...]) + b2_ref[0]


def kernel(query, W0, b0, W1, b1, W2, b2):
    B, S, D_IN = query.shape
    D_H = W0.shape[1]
    D_OUT = W2.shape[1]
    BS = min(2048, S)

    b0r = b0[:, None, :]
    b1r = b1[:, None, :]
    b2r = b2[:, None, :]

    return pl.pallas_call(
        _mlp_kernel,
        out_shape=jax.ShapeDtypeStruct((B, S, D_OUT), query.dtype),
        grid=(B, S // BS),
        in_specs=[
            pl.BlockSpec((1, BS, D_IN), lambda b, s: (b, s, 0)),
            pl.BlockSpec((1, D_H, D_IN), lambda b, s: (b, 0, 0)),
            pl.BlockSpec((1, 1, D_H), lambda b, s: (b, 0, 0)),
            pl.BlockSpec((1, D_H, D_H), lambda b, s: (b, 0, 0)),
            pl.BlockSpec((1, 1, D_H), lambda b, s: (b, 0, 0)),
            pl.BlockSpec((1, D_OUT, D_H), lambda b, s: (b, 0, 0)),
            pl.BlockSpec((1, 1, D_OUT), lambda b, s: (b, 0, 0)),
        ],
        out_specs=pl.BlockSpec((1, BS, D_OUT), lambda b, s: (b, s, 0)),
        scratch_shapes=[
            pltpu.VMEM((D_H, D_IN), jnp.bfloat16),
            pltpu.VMEM((D_H, D_H), jnp.bfloat16),
            pltpu.VMEM((D_OUT, D_H), jnp.bfloat16),
        ],
        compiler_params=pltpu.CompilerParams(
            dimension_semantics=("parallel", "arbitrary"),
            vmem_limit_bytes=56 * 1024 * 1024,
        ),
        name="ltm_mlp",
    )(query, W0, b0r, W1, b1r, W2, b2r)


# speedup vs baseline: 1.1885x; 1.0152x over previous
import jax
import jax.numpy as jnp
from jax.experimental import pallas as pl
from jax.experimental.pallas import tpu as pltpu


def _nt_dot(x, w):
    return jax.lax.dot_general(
        x, w, (((1,), (1,)), ((), ())), preferred_element_type=jnp.float32
    )


def _mlp_kernel(x_ref, w0_ref, b0_ref, w1_ref, b1_ref, w2_ref, b2_ref, o_ref):
    zero = jnp.bfloat16(0)
    w0 = w0_ref[0].astype(jnp.bfloat16)
    w1 = w1_ref[0].astype(jnp.bfloat16)
    w2 = w2_ref[0].astype(jnp.bfloat16)
    x = x_ref[0].astype(jnp.bfloat16)
    h = jnp.maximum((_nt_dot(x, w0) + b0_ref[0]).astype(jnp.bfloat16), zero)
    h = jnp.maximum((_nt_dot(h, w1) + b1_ref[0]).astype(jnp.bfloat16), zero)
    o_ref[0] = _nt_dot(h, w2) + b2_ref[0]


def kernel(query, W0, b0, W1, b1, W2, b2):
    B, S, D_IN = query.shape
    D_H = W0.shape[1]
    D_OUT = W2.shape[1]

    b0r = b0[:, None, :]
    b1r = b1[:, None, :]
    b2r = b2[:, None, :]

    BS = min(2048, S)
    return pl.pallas_call(
        _mlp_kernel,
        out_shape=jax.ShapeDtypeStruct((B, S, D_OUT), query.dtype),
        grid=(B, S // BS),
        in_specs=[
            pl.BlockSpec((1, BS, D_IN), lambda b, s: (b, s, 0)),
            pl.BlockSpec((1, D_H, D_IN), lambda b, s: (b, 0, 0)),
            pl.BlockSpec((1, 1, D_H), lambda b, s: (b, 0, 0)),
            pl.BlockSpec((1, D_H, D_H), lambda b, s: (b, 0, 0)),
            pl.BlockSpec((1, 1, D_H), lambda b, s: (b, 0, 0)),
            pl.BlockSpec((1, D_OUT, D_H), lambda b, s: (b, 0, 0)),
            pl.BlockSpec((1, 1, D_OUT), lambda b, s: (b, 0, 0)),
        ],
        out_specs=pl.BlockSpec((1, BS, D_OUT), lambda b, s: (b, s, 0)),
        compiler_params=pltpu.CompilerParams(
            dimension_semantics=("parallel", "arbitrary"),
            vmem_limit_bytes=56 * 1024 * 1024,
        ),
        name="ltm_mlp",
    )(query, W0, b0r, W1, b1r, W2, b2r)
